# baseline (speedup 1.0000x reference)
import dataclasses
import functools

import jax
import jax.numpy as jnp
from jax import lax
from jax.experimental import pallas as pl
from jax.experimental.pallas import tpu as pltpu
from jax.experimental.pallas import tpu_sc as plsc

N_NODES = 10000
D_IN = 256
D_OUT = 256
HALF = 128
LANES = 16

NC = 2
NS = 16

CHUNK = 64
N_CHUNKS = 160
E_PAD = NS * N_CHUNKS * CHUNK
N_DGROUPS = E_PAD // CHUNK
DUMMY_ROW = N_NODES
SLICE = 632
ACC_ROWS = NS * SLICE
CRANGE = 320
CNT_ROWS = NC * NS * CRANGE

ROW_BLOCK = 1000


def _sc_aggregate(x_stack, src_all, dst_flat):
    mesh = plsc.VectorSubcoreMesh(core_axis_name="c", subcore_axis_name="s")
    cp = pltpu.CompilerParams()
    if "needs_layout_passes" in pltpu.CompilerParams.__dataclass_fields__:
        cp = dataclasses.replace(cp, needs_layout_passes=False)

    @functools.partial(
        pl.kernel,
        mesh=mesh,
        compiler_params=cp,
        out_type=[
            jax.ShapeDtypeStruct((2 * ACC_ROWS, HALF), jnp.float32),
            jax.ShapeDtypeStruct((CNT_ROWS, LANES), jnp.float32),
        ],
        scratch_types=[
            pltpu.VMEM((CHUNK,), jnp.int32),
            pltpu.VMEM((CHUNK,), jnp.int32),
            pltpu.VMEM((CHUNK, HALF), jnp.float32),
            pltpu.VMEM((CRANGE, LANES), jnp.float32),
            pltpu.VMEM_SHARED((ACC_ROWS, HALF), jnp.float32),
            pltpu.SemaphoreType.DMA,
        ],
    )
    def k(x_hbm, src_hbm, dst_hbm, out_hbm, cnt_hbm,
          src_v, dst_v, rows_v, cnt_v, acc_s, sem):
        c = lax.axis_index("c")
        s = lax.axis_index("s")
        wid = c * NS + s
        base = pl.multiple_of(s * SLICE, 8)
        obase = pl.multiple_of(c * ACC_ROWS + base, 8)

        zero16 = jnp.zeros((LANES,), jnp.float32)

        @pl.loop(0, CHUNK)
        def _(i):
            @pl.loop(0, HALF, step=LANES)
            def _(j):
                rows_v[i, pl.ds(j, LANES)] = zero16

        @pl.loop(0, CRANGE)
        def _(i):
            cnt_v[i, pl.ds(0, LANES)] = zero16

        for t in range(SLICE // CHUNK):
            pltpu.sync_copy(rows_v, acc_s.at[pl.ds(base + t * CHUNK, CHUNK)])
        pltpu.sync_copy(rows_v.at[pl.ds(0, SLICE % CHUNK)],
                        acc_s.at[pl.ds(base + SLICE - SLICE % CHUNK,
                                       SLICE % CHUNK)])

        plsc.subcore_barrier()

        lane_iota = lax.iota(jnp.int32, LANES)
        one16 = jnp.ones((LANES,), jnp.float32)
        lo = wid * CRANGE

        @pl.loop(0, N_DGROUPS)
        def _(g):
            goff = pl.multiple_of(g * CHUNK, CHUNK)
            pltpu.sync_copy(dst_hbm.at[pl.ds(goff, CHUNK)], dst_v)
            for kk in range(CHUNK // LANES):
                val = dst_v[pl.ds(kk * LANES, LANES)]
                rel = plsc.bitcast(val - lo, jnp.uint32)
                mask = rel < jnp.uint32(CRANGE)
                row = plsc.bitcast(
                    jnp.minimum(rel, jnp.uint32(CRANGE - 1)), jnp.int32)
                plsc.addupdate_scatter(cnt_v, [row, lane_iota], one16,
                                       mask=mask)

        cbase = pl.multiple_of(wid * CRANGE, 8)
        pltpu.sync_copy(cnt_v, cnt_hbm.at[pl.ds(cbase, CRANGE)])

        per_w = N_CHUNKS * CHUNK

        @pl.loop(0, N_CHUNKS)
        def _(g):
            off = pl.multiple_of(wid * per_w + g * CHUNK, CHUNK)
            soff = pl.multiple_of(s * per_w + g * CHUNK, CHUNK)
            pltpu.sync_copy(src_hbm.at[pl.ds(off, CHUNK)], src_v)
            pltpu.sync_copy(dst_hbm.at[pl.ds(soff, CHUNK)], dst_v)
            pltpu.async_copy(x_hbm.at[src_v], rows_v, sem).wait()
            pltpu.sync_copy(rows_v, acc_s.at[dst_v], add=True)

        plsc.subcore_barrier()

        for t in range(SLICE // CHUNK):
            pltpu.sync_copy(acc_s.at[pl.ds(base + t * CHUNK, CHUNK)], rows_v)
            pltpu.sync_copy(rows_v, out_hbm.at[pl.ds(obase + t * CHUNK, CHUNK)])
        rem = SLICE % CHUNK
        pltpu.sync_copy(acc_s.at[pl.ds(base + SLICE - rem, rem)],
                        rows_v.at[pl.ds(0, rem)])
        pltpu.sync_copy(rows_v.at[pl.ds(0, rem)],
                        out_hbm.at[pl.ds(obase + SLICE - rem, rem)])

    return k(x_stack, src_all, dst_flat)


def _tc_body(slo_ref, shi_ref, cnt_ref, x_ref, wlo_ref, whi_ref, wr_ref,
             b_ref, o_ref):
    cnt = jnp.maximum(jnp.sum(cnt_ref[...], axis=1, keepdims=True), 1.0)
    a_lo = slo_ref[...] / cnt
    a_hi = shi_ref[...] / cnt
    acc = jnp.dot(a_lo, wlo_ref[...], preferred_element_type=jnp.float32)
    acc += jnp.dot(a_hi, whi_ref[...], preferred_element_type=jnp.float32)
    acc += jnp.dot(x_ref[...], wr_ref[...], preferred_element_type=jnp.float32)
    acc += b_ref[...]
    o_ref[...] = jnp.maximum(acc, 0.0)


def _tc_combine(sums_lo, sums_hi, cnt16, x, wlo_t, whi_t, wr_t, b2d):
    grid = (N_NODES // ROW_BLOCK,)
    return pl.pallas_call(
        _tc_body,
        grid=grid,
        in_specs=[
            pl.BlockSpec((ROW_BLOCK, HALF), lambda i: (i, 0)),
            pl.BlockSpec((ROW_BLOCK, HALF), lambda i: (i, 0)),
            pl.BlockSpec((ROW_BLOCK, LANES), lambda i: (i, 0)),
            pl.BlockSpec((ROW_BLOCK, D_IN), lambda i: (i, 0)),
            pl.BlockSpec((HALF, D_OUT), lambda i: (0, 0)),
            pl.BlockSpec((HALF, D_OUT), lambda i: (0, 0)),
            pl.BlockSpec((D_IN, D_OUT), lambda i: (0, 0)),
            pl.BlockSpec((1, D_OUT), lambda i: (0, 0)),
        ],
        out_specs=pl.BlockSpec((ROW_BLOCK, D_OUT), lambda i: (i, 0)),
        out_shape=jax.ShapeDtypeStruct((N_NODES, D_OUT), jnp.float32),
    )(sums_lo, sums_hi, cnt16, x, wlo_t, whi_t, wr_t, b2d)


def kernel(x, edge_index, W_l, b_l, W_r):
    n_edges = edge_index.shape[1]
    pad = E_PAD - n_edges
    src = edge_index[0].astype(jnp.int32)
    dst = edge_index[1].astype(jnp.int32)
    src_pad = jnp.concatenate([src, jnp.zeros((pad,), jnp.int32)])
    dst_pad = jnp.concatenate([dst, jnp.full((pad,), DUMMY_ROW, jnp.int32)])
    src_all = jnp.concatenate([src_pad, src_pad + N_NODES])
    x_stack = jnp.concatenate([x[:, :HALF], x[:, HALF:]], axis=0)

    out_stack, cnt_out = _sc_aggregate(x_stack, src_all, dst_pad)
    sums_lo = out_stack[:N_NODES]
    sums_hi = out_stack[ACC_ROWS:ACC_ROWS + N_NODES]
    cnt16 = cnt_out[:N_NODES]

    wlo_t = W_l[:, :HALF].T
    whi_t = W_l[:, HALF:].T
    wr_t = W_r.T
    b2d = b_l.reshape(1, D_OUT)
    return _tc_combine(sums_lo, sums_hi, cnt16, x, wlo_t, whi_t, wr_t, b2d)


# speedup vs baseline: 1.9773x; 1.9773x over previous
import dataclasses
import functools

import jax
import jax.numpy as jnp
from jax import lax
from jax.experimental import pallas as pl
from jax.experimental.pallas import tpu as pltpu
from jax.experimental.pallas import tpu_sc as plsc

N_NODES = 10000
D_IN = 256
D_OUT = 256
HALF = 128
LANES = 16

NC = 2
NS = 16

CHUNK = 64
N_CHUNKS = 160
E_PAD = NS * N_CHUNKS * CHUNK
N_DGROUPS = E_PAD // CHUNK
DUMMY_ROW = N_NODES
SLICE = 632
ACC_ROWS = NS * SLICE
CRANGE = 320
CNT_ROWS = NC * NS * CRANGE

ROW_BLOCK = 1000


def _sc_aggregate(x_stack, src_all, dst_flat):
    mesh = plsc.VectorSubcoreMesh(core_axis_name="c", subcore_axis_name="s")
    cp = pltpu.CompilerParams()
    if "needs_layout_passes" in pltpu.CompilerParams.__dataclass_fields__:
        cp = dataclasses.replace(cp, needs_layout_passes=False)

    @functools.partial(
        pl.kernel,
        mesh=mesh,
        compiler_params=cp,
        out_type=[
            jax.ShapeDtypeStruct((2 * ACC_ROWS, HALF), jnp.float32),
            jax.ShapeDtypeStruct((CNT_ROWS, LANES), jnp.float32),
        ],
        scratch_types=[
            pltpu.VMEM((CHUNK,), jnp.int32),
            pltpu.VMEM((CHUNK,), jnp.int32),
            pltpu.VMEM((CHUNK, HALF), jnp.float32),
            pltpu.VMEM((CRANGE, LANES), jnp.float32),
            pltpu.VMEM((512,), jnp.int32),
            pltpu.VMEM_SHARED((ACC_ROWS, HALF), jnp.float32),
            pltpu.SemaphoreType.DMA,
        ],
    )
    def k(x_hbm, src_hbm, dst_hbm, out_hbm, cnt_hbm,
          src_v, dst_v, rows_v, cnt_v, dstc_v, acc_s, sem):
        c = lax.axis_index("c")
        s = lax.axis_index("s")
        wid = c * NS + s
        base = pl.multiple_of(s * SLICE, 8)
        obase = pl.multiple_of(c * ACC_ROWS + base, 8)

        zero16 = jnp.zeros((LANES,), jnp.float32)

        @pl.loop(0, CHUNK)
        def _(i):
            @pl.loop(0, HALF, step=LANES)
            def _(j):
                rows_v[i, pl.ds(j, LANES)] = zero16

        @pl.loop(0, CRANGE)
        def _(i):
            cnt_v[i, pl.ds(0, LANES)] = zero16

        for t in range(SLICE // CHUNK):
            pltpu.sync_copy(rows_v, acc_s.at[pl.ds(base + t * CHUNK, CHUNK)])
        pltpu.sync_copy(rows_v.at[pl.ds(0, SLICE % CHUNK)],
                        acc_s.at[pl.ds(base + SLICE - SLICE % CHUNK,
                                       SLICE % CHUNK)])

        plsc.subcore_barrier()

        lane_iota = lax.iota(jnp.int32, LANES)
        one16 = jnp.ones((LANES,), jnp.float32)
        lo = wid * CRANGE

        @pl.loop(0, E_PAD // 512)
        def _(g):
            goff = pl.multiple_of(g * 512, 512)
            pltpu.sync_copy(dst_hbm.at[pl.ds(goff, 512)], dstc_v)

            @pl.loop(0, 512, step=LANES)
            def _(kk):
                val = dstc_v[pl.ds(kk, LANES)]
                rel = plsc.bitcast(val - lo, jnp.uint32)
                mask = rel < jnp.uint32(CRANGE)
                row = plsc.bitcast(
                    jnp.minimum(rel, jnp.uint32(CRANGE - 1)), jnp.int32)
                plsc.addupdate_scatter(cnt_v, [row, lane_iota], one16,
                                       mask=mask)


        cbase = pl.multiple_of(wid * CRANGE, 8)
        pltpu.sync_copy(cnt_v, cnt_hbm.at[pl.ds(cbase, CRANGE)])

        per_w = N_CHUNKS * CHUNK

        @pl.loop(0, N_CHUNKS)
        def _(g):
            off = pl.multiple_of(wid * per_w + g * CHUNK, CHUNK)
            soff = pl.multiple_of(s * per_w + g * CHUNK, CHUNK)
            pltpu.sync_copy(src_hbm.at[pl.ds(off, CHUNK)], src_v)
            pltpu.sync_copy(dst_hbm.at[pl.ds(soff, CHUNK)], dst_v)
            pltpu.async_copy(x_hbm.at[src_v], rows_v, sem).wait()
            pltpu.sync_copy(rows_v, acc_s.at[dst_v], add=True)

        plsc.subcore_barrier()

        pltpu.sync_copy(acc_s.at[pl.ds(base, SLICE)],
                        out_hbm.at[pl.ds(obase, SLICE)])

    return k(x_stack, src_all, dst_flat)


def _tc_body(slo_ref, shi_ref, cnt_ref, x_ref, wlo_ref, whi_ref, wr_ref,
             b_ref, o_ref):
    cnt = jnp.maximum(jnp.sum(cnt_ref[...], axis=1, keepdims=True), 1.0)
    a_lo = slo_ref[...] / cnt
    a_hi = shi_ref[...] / cnt
    acc = jnp.dot(a_lo, wlo_ref[...], preferred_element_type=jnp.float32)
    acc += jnp.dot(a_hi, whi_ref[...], preferred_element_type=jnp.float32)
    acc += jnp.dot(x_ref[...], wr_ref[...], preferred_element_type=jnp.float32)
    acc += b_ref[...]
    o_ref[...] = jnp.maximum(acc, 0.0)


def _tc_combine(sums_lo, sums_hi, cnt16, x, wlo_t, whi_t, wr_t, b2d):
    grid = (N_NODES // ROW_BLOCK,)
    return pl.pallas_call(
        _tc_body,
        grid=grid,
        in_specs=[
            pl.BlockSpec((ROW_BLOCK, HALF), lambda i: (i, 0)),
            pl.BlockSpec((ROW_BLOCK, HALF), lambda i: (i, 0)),
            pl.BlockSpec((ROW_BLOCK, LANES), lambda i: (i, 0)),
            pl.BlockSpec((ROW_BLOCK, D_IN), lambda i: (i, 0)),
            pl.BlockSpec((HALF, D_OUT), lambda i: (0, 0)),
            pl.BlockSpec((HALF, D_OUT), lambda i: (0, 0)),
            pl.BlockSpec((D_IN, D_OUT), lambda i: (0, 0)),
            pl.BlockSpec((1, D_OUT), lambda i: (0, 0)),
        ],
        out_specs=pl.BlockSpec((ROW_BLOCK, D_OUT), lambda i: (i, 0)),
        out_shape=jax.ShapeDtypeStruct((N_NODES, D_OUT), jnp.float32),
    )(sums_lo, sums_hi, cnt16, x, wlo_t, whi_t, wr_t, b2d)


def kernel(x, edge_index, W_l, b_l, W_r):
    n_edges = edge_index.shape[1]
    pad = E_PAD - n_edges
    src = edge_index[0].astype(jnp.int32)
    dst = edge_index[1].astype(jnp.int32)
    src_pad = jnp.concatenate([src, jnp.zeros((pad,), jnp.int32)])
    dst_pad = jnp.concatenate([dst, jnp.full((pad,), DUMMY_ROW, jnp.int32)])
    src_all = jnp.concatenate([src_pad, src_pad + N_NODES])
    x_stack = jnp.concatenate([x[:, :HALF], x[:, HALF:]], axis=0)

    out_stack, cnt_out = _sc_aggregate(x_stack, src_all, dst_pad)
    sums_lo = out_stack[:N_NODES]
    sums_hi = out_stack[ACC_ROWS:ACC_ROWS + N_NODES]
    cnt16 = cnt_out[:N_NODES]

    wlo_t = W_l[:, :HALF].T
    whi_t = W_l[:, HALF:].T
    wr_t = W_r.T
    b2d = b_l.reshape(1, D_OUT)
    return _tc_combine(sums_lo, sums_hi, cnt16, x, wlo_t, whi_t, wr_t, b2d)


# speedup vs baseline: 3.3952x; 1.7171x over previous
import dataclasses
import functools

import jax
import jax.numpy as jnp
from jax import lax
from jax.experimental import pallas as pl
from jax.experimental.pallas import tpu as pltpu
from jax.experimental.pallas import tpu_sc as plsc

N_NODES = 10000
D_IN = 256
D_OUT = 256
HALF = 128
LANES = 16

NC = 2
NS = 16

CHUNK = 32
N_CHUNKS = 320
E_PAD = NS * N_CHUNKS * CHUNK
N_DGROUPS = E_PAD // CHUNK
DUMMY_ROW = N_NODES
SLICE = 632
ACC_ROWS = NS * SLICE
HIST_CB = 128

ROW_BLOCK = 1000


def _sc_aggregate(x_stack, src_all, dst_flat):
    mesh = plsc.VectorSubcoreMesh(core_axis_name="c", subcore_axis_name="s")
    cp = pltpu.CompilerParams()
    if "needs_layout_passes" in pltpu.CompilerParams.__dataclass_fields__:
        cp = dataclasses.replace(cp, needs_layout_passes=False)

    @functools.partial(
        pl.kernel,
        mesh=mesh,
        compiler_params=cp,
        out_type=jax.ShapeDtypeStruct((2 * ACC_ROWS, HALF), jnp.float32),
        scratch_types=[
            pltpu.VMEM((2, CHUNK), jnp.int32),
            pltpu.VMEM((2, CHUNK), jnp.int32),
            pltpu.VMEM((2, CHUNK), jnp.int32),
            pltpu.VMEM((2, CHUNK), jnp.int32),
            pltpu.VMEM((CHUNK, HALF), jnp.float32),
            pltpu.VMEM((CHUNK, HALF), jnp.float32),
            pltpu.VMEM_SHARED((ACC_ROWS, HALF), jnp.float32),
            pltpu.SemaphoreType.DMA,
            pltpu.SemaphoreType.DMA,
            pltpu.SemaphoreType.DMA,
            pltpu.SemaphoreType.DMA,
            pltpu.SemaphoreType.DMA,
            pltpu.SemaphoreType.DMA,
            pltpu.SemaphoreType.DMA,
            pltpu.SemaphoreType.DMA,
        ],
    )
    def k(x_hbm, src_hbm, dst_hbm, out_hbm,
          sd0, sd1, sd2, sd3, rows0, rows1, acc_s,
          sg0, sg1, ss0, ss1, si0, si1, si2, si3):
        sd = (sd0, sd1, sd2, sd3)
        rows = (rows0, rows1)
        sg = (sg0, sg1)
        ss = (ss0, ss1)
        si = (si0, si1, si2, si3)
        c = lax.axis_index("c")
        s = lax.axis_index("s")
        base = pl.multiple_of(s * SLICE, 8)
        obase = pl.multiple_of(c * ACC_ROWS + base, 8)
        per_w = N_CHUNKS * CHUNK
        src_base = c * E_PAD + s * per_w
        dst_base = s * per_w

        def idx_issue(jj, off_dyn):
            pltpu.async_copy(src_hbm.at[pl.ds(src_base + off_dyn, CHUNK)],
                             sd[jj].at[0], si[jj])
            pltpu.async_copy(dst_hbm.at[pl.ds(dst_base + off_dyn, CHUNK)],
                             sd[jj].at[1], si[jj])

        def idx_wait(jj):
            pltpu.make_async_copy(src_hbm.at[pl.ds(src_base, CHUNK)],
                                  sd[jj].at[0], si[jj]).wait()
            pltpu.make_async_copy(dst_hbm.at[pl.ds(dst_base, CHUNK)],
                                  sd[jj].at[1], si[jj]).wait()

        def gather_issue(jj, p):
            pltpu.async_copy(x_hbm.at[sd[jj].at[0]], rows[p], sg[p])

        def gather_wait(jj, p):
            pltpu.make_async_copy(x_hbm.at[sd[jj].at[0]], rows[p],
                                  sg[p]).wait()

        def scatter_issue(jj, p):
            pltpu.async_copy(rows[p], acc_s.at[sd[jj].at[1]], ss[p], add=True)

        def scatter_wait(jj, p):
            pltpu.make_async_copy(rows[p], acc_s.at[sd[jj].at[1]],
                                  ss[p]).wait()

        zero16 = jnp.zeros((LANES,), jnp.float32)

        @pl.loop(0, CHUNK)
        def _(i):
            @pl.loop(0, HALF, step=LANES)
            def _(j):
                rows0[i, pl.ds(j, LANES)] = zero16

        for t in range(SLICE // CHUNK):
            pltpu.sync_copy(rows0, acc_s.at[pl.ds(base + t * CHUNK, CHUNK)])
        rem = SLICE % CHUNK
        pltpu.sync_copy(rows0.at[pl.ds(0, rem)],
                        acc_s.at[pl.ds(base + SLICE - rem, rem)])

        plsc.subcore_barrier()

        idx_issue(0, 0)
        idx_issue(1, CHUNK)
        idx_wait(0)
        gather_issue(0, 0)

        @pl.loop(0, N_CHUNKS // 4)
        def _(it):
            t0 = it * 4
            for j in range(4):
                p = j % 2
                t_dyn = t0 + j
                gather_wait(j, p)
                scatter_issue(j, p)

                @pl.when(t_dyn + 2 < N_CHUNKS)
                def _(jj=(j + 2) % 4, t_dyn=t_dyn):
                    idx_issue(jj, (t_dyn + 2) * CHUNK)

                @pl.when(t_dyn + 1 < N_CHUNKS)
                def _(j=j, p=p, t_dyn=t_dyn):
                    @pl.when(t_dyn >= 1)
                    def _():
                        scatter_wait((j + 3) % 4, 1 - p)

                    idx_wait((j + 1) % 4)
                    gather_issue((j + 1) % 4, 1 - p)

        scatter_wait(2, 0)
        scatter_wait(3, 1)

        plsc.subcore_barrier()

        pltpu.sync_copy(acc_s.at[pl.ds(base, SLICE)],
                        out_hbm.at[pl.ds(obase, SLICE)])

    return k(x_stack, src_all, dst_flat)


def _hist_body(dst_ref, o_ref):
    @pl.when(pl.program_id(0) == 0)
    def _():
        o_ref[...] = jnp.zeros((HALF, HALF), jnp.float32)

    blk = dst_ref[...]
    lane = jax.lax.broadcasted_iota(jnp.int32, (1, HALF), 1)
    acc = jnp.zeros((HALF, HALF), jnp.float32)
    for b in range(HIST_CB):
        col = blk[:, b:b + 1]
        ohl = (jnp.bitwise_and(col, HALF - 1) == lane).astype(jnp.float32)
        ohh = (jax.lax.shift_right_logical(col, 7) == lane).astype(jnp.float32)
        acc += jax.lax.dot_general(ohh, ohl, (((0,), (0,)), ((), ())),
                                   preferred_element_type=jnp.float32)
    o_ref[...] += acc


def _tc_hist(dst_t):
    grid = (E_PAD // HALF // HIST_CB,)
    return pl.pallas_call(
        _hist_body,
        grid=grid,
        in_specs=[pl.BlockSpec((HALF, HIST_CB), lambda i: (0, i))],
        out_specs=pl.BlockSpec((HALF, HALF), lambda i: (0, 0)),
        out_shape=jax.ShapeDtypeStruct((HALF, HALF), jnp.float32),
    )(dst_t)


def _tc_body(slo_ref, shi_ref, cnt_ref, x_ref, wlo_ref, whi_ref, wr_ref,
             b_ref, o_ref):
    cnt = jnp.maximum(cnt_ref[...], 1.0)
    a_lo = slo_ref[...] / cnt
    a_hi = shi_ref[...] / cnt
    acc = jnp.dot(a_lo, wlo_ref[...], preferred_element_type=jnp.float32)
    acc += jnp.dot(a_hi, whi_ref[...], preferred_element_type=jnp.float32)
    acc += jnp.dot(x_ref[...], wr_ref[...], preferred_element_type=jnp.float32)
    acc += b_ref[...]
    o_ref[...] = jnp.maximum(acc, 0.0)


def _tc_combine(sums_lo, sums_hi, cnt16, x, wlo_t, whi_t, wr_t, b2d):
    grid = (N_NODES // ROW_BLOCK,)
    return pl.pallas_call(
        _tc_body,
        grid=grid,
        in_specs=[
            pl.BlockSpec((ROW_BLOCK, HALF), lambda i: (i, 0)),
            pl.BlockSpec((ROW_BLOCK, HALF), lambda i: (i, 0)),
            pl.BlockSpec((ROW_BLOCK, 1), lambda i: (i, 0)),
            pl.BlockSpec((ROW_BLOCK, D_IN), lambda i: (i, 0)),
            pl.BlockSpec((HALF, D_OUT), lambda i: (0, 0)),
            pl.BlockSpec((HALF, D_OUT), lambda i: (0, 0)),
            pl.BlockSpec((D_IN, D_OUT), lambda i: (0, 0)),
            pl.BlockSpec((1, D_OUT), lambda i: (0, 0)),
        ],
        out_specs=pl.BlockSpec((ROW_BLOCK, D_OUT), lambda i: (i, 0)),
        out_shape=jax.ShapeDtypeStruct((N_NODES, D_OUT), jnp.float32),
    )(sums_lo, sums_hi, cnt16, x, wlo_t, whi_t, wr_t, b2d)


def kernel(x, edge_index, W_l, b_l, W_r):
    n_edges = edge_index.shape[1]
    pad = E_PAD - n_edges
    src = edge_index[0].astype(jnp.int32)
    dst = edge_index[1].astype(jnp.int32)
    src_pad = jnp.concatenate([src, jnp.zeros((pad,), jnp.int32)])
    dst_pad = jnp.concatenate([dst, jnp.full((pad,), DUMMY_ROW, jnp.int32)])
    src_all = jnp.concatenate([src_pad, src_pad + N_NODES])
    x_stack = jnp.concatenate([x[:, :HALF], x[:, HALF:]], axis=0)

    dst_t = dst_pad.reshape(E_PAD // HALF, HALF).T
    cnt2d = _tc_hist(dst_t)
    cnt_col = cnt2d.reshape(-1)[:N_NODES].reshape(N_NODES, 1)
    out_stack = _sc_aggregate(x_stack, src_all, dst_pad)
    sums_lo = out_stack[:N_NODES]
    sums_hi = out_stack[ACC_ROWS:ACC_ROWS + N_NODES]

    wlo_t = W_l[:, :HALF].T
    whi_t = W_l[:, HALF:].T
    wr_t = W_r.T
    b2d = b_l.reshape(1, D_OUT)
    return _tc_combine(sums_lo, sums_hi, cnt_col, x, wlo_t, whi_t, wr_t, b2d)


# speedup vs baseline: 3.6302x; 1.0692x over previous
import dataclasses
import functools

import jax
import jax.numpy as jnp
from jax import lax
from jax.experimental import pallas as pl
from jax.experimental.pallas import tpu as pltpu
from jax.experimental.pallas import tpu_sc as plsc

N_NODES = 10000
D_IN = 256
D_OUT = 256
HALF = 128
LANES = 16

NC = 2
NS = 16

CHUNK = 40
N_CHUNKS = 256
E_PAD = NS * N_CHUNKS * CHUNK
N_DGROUPS = E_PAD // CHUNK
DUMMY_ROW = N_NODES
SLICE = 632
ACC_ROWS = NS * SLICE
HIST_CB = 128

ROW_BLOCK = 1000


def _sc_aggregate(x_stack, src_all, dst_flat):
    mesh = plsc.VectorSubcoreMesh(core_axis_name="c", subcore_axis_name="s")
    cp = pltpu.CompilerParams()
    if "needs_layout_passes" in pltpu.CompilerParams.__dataclass_fields__:
        cp = dataclasses.replace(cp, needs_layout_passes=False)

    @functools.partial(
        pl.kernel,
        mesh=mesh,
        compiler_params=cp,
        out_type=jax.ShapeDtypeStruct((2 * ACC_ROWS, HALF), jnp.float32),
        scratch_types=[
            pltpu.VMEM((2, CHUNK), jnp.int32),
            pltpu.VMEM((2, CHUNK), jnp.int32),
            pltpu.VMEM((2, CHUNK), jnp.int32),
            pltpu.VMEM((2, CHUNK), jnp.int32),
            pltpu.VMEM((CHUNK, HALF), jnp.float32),
            pltpu.VMEM((CHUNK, HALF), jnp.float32),
            pltpu.VMEM_SHARED((ACC_ROWS, HALF), jnp.float32),
            pltpu.SemaphoreType.DMA,
            pltpu.SemaphoreType.DMA,
            pltpu.SemaphoreType.DMA,
            pltpu.SemaphoreType.DMA,
            pltpu.SemaphoreType.DMA,
            pltpu.SemaphoreType.DMA,
            pltpu.SemaphoreType.DMA,
            pltpu.SemaphoreType.DMA,
        ],
    )
    def k(x_hbm, src_hbm, dst_hbm, out_hbm,
          sd0, sd1, sd2, sd3, rows0, rows1, acc_s,
          sg0, sg1, ss0, ss1, si0, si1, si2, si3):
        sd = (sd0, sd1, sd2, sd3)
        rows = (rows0, rows1)
        sg = (sg0, sg1)
        ss = (ss0, ss1)
        si = (si0, si1, si2, si3)
        c = lax.axis_index("c")
        s = lax.axis_index("s")
        base = pl.multiple_of(s * SLICE, 8)
        obase = pl.multiple_of(c * ACC_ROWS + base, 8)
        per_w = N_CHUNKS * CHUNK
        src_base = c * E_PAD + s * per_w
        dst_base = s * per_w

        def idx_issue(jj, off_dyn):
            pltpu.async_copy(src_hbm.at[pl.ds(src_base + off_dyn, CHUNK)],
                             sd[jj].at[0], si[jj])
            pltpu.async_copy(dst_hbm.at[pl.ds(dst_base + off_dyn, CHUNK)],
                             sd[jj].at[1], si[jj])

        def idx_wait(jj):
            pltpu.make_async_copy(src_hbm.at[pl.ds(src_base, CHUNK)],
                                  sd[jj].at[0], si[jj]).wait()
            pltpu.make_async_copy(dst_hbm.at[pl.ds(dst_base, CHUNK)],
                                  sd[jj].at[1], si[jj]).wait()

        def gather_issue(jj, p):
            pltpu.async_copy(x_hbm.at[sd[jj].at[0]], rows[p], sg[p])

        def gather_wait(jj, p):
            pltpu.make_async_copy(x_hbm.at[sd[jj].at[0]], rows[p],
                                  sg[p]).wait()

        def scatter_issue(jj, p):
            pltpu.async_copy(rows[p], acc_s.at[sd[jj].at[1]], ss[p], add=True)

        def scatter_wait(jj, p):
            pltpu.make_async_copy(rows[p], acc_s.at[sd[jj].at[1]],
                                  ss[p]).wait()

        zero16 = jnp.zeros((LANES,), jnp.float32)

        @pl.loop(0, CHUNK)
        def _(i):
            @pl.loop(0, HALF, step=LANES)
            def _(j):
                rows0[i, pl.ds(j, LANES)] = zero16

        for t in range(SLICE // CHUNK):
            pltpu.sync_copy(rows0, acc_s.at[pl.ds(base + t * CHUNK, CHUNK)])
        rem = SLICE % CHUNK
        pltpu.sync_copy(rows0.at[pl.ds(0, rem)],
                        acc_s.at[pl.ds(base + SLICE - rem, rem)])

        plsc.subcore_barrier()

        idx_issue(0, 0)
        idx_issue(1, CHUNK)
        idx_wait(0)
        gather_issue(0, 0)

        @pl.loop(0, N_CHUNKS // 4)
        def _(it):
            t0 = it * 4
            for j in range(4):
                p = j % 2
                t_dyn = t0 + j
                gather_wait(j, p)
                scatter_issue(j, p)

                @pl.when(t_dyn + 2 < N_CHUNKS)
                def _(jj=(j + 2) % 4, t_dyn=t_dyn):
                    idx_issue(jj, (t_dyn + 2) * CHUNK)

                @pl.when(t_dyn + 1 < N_CHUNKS)
                def _(j=j, p=p, t_dyn=t_dyn):
                    @pl.when(t_dyn >= 1)
                    def _():
                        scatter_wait((j + 3) % 4, 1 - p)

                    idx_wait((j + 1) % 4)
                    gather_issue((j + 1) % 4, 1 - p)

        scatter_wait(2, 0)
        scatter_wait(3, 1)

        plsc.subcore_barrier()

        pltpu.sync_copy(acc_s.at[pl.ds(base, SLICE)],
                        out_hbm.at[pl.ds(obase, SLICE)])

    return k(x_stack, src_all, dst_flat)


def _hist_body(dst_ref, o_ref):
    @pl.when(pl.program_id(0) == 0)
    def _():
        o_ref[...] = jnp.zeros((HALF, HALF), jnp.float32)

    blk = dst_ref[...]
    lane = jax.lax.broadcasted_iota(jnp.int32, (1, HALF), 1)
    acc = jnp.zeros((HALF, HALF), jnp.float32)
    for b in range(HIST_CB):
        col = blk[:, b:b + 1]
        ohl = (jnp.bitwise_and(col, HALF - 1) == lane).astype(jnp.float32)
        ohh = (jax.lax.shift_right_logical(col, 7) == lane).astype(jnp.float32)
        acc += jax.lax.dot_general(ohh, ohl, (((0,), (0,)), ((), ())),
                                   preferred_element_type=jnp.float32)
    o_ref[...] += acc


def _tc_hist(dst_t):
    grid = (E_PAD // HALF // HIST_CB,)
    return pl.pallas_call(
        _hist_body,
        grid=grid,
        in_specs=[pl.BlockSpec((HALF, HIST_CB), lambda i: (0, i))],
        out_specs=pl.BlockSpec((HALF, HALF), lambda i: (0, 0)),
        out_shape=jax.ShapeDtypeStruct((HALF, HALF), jnp.float32),
    )(dst_t)


def _tc_body(slo_ref, shi_ref, cnt_ref, x_ref, wlo_ref, whi_ref, wr_ref,
             b_ref, o_ref):
    cnt = jnp.maximum(cnt_ref[...], 1.0)
    a_lo = slo_ref[...] / cnt
    a_hi = shi_ref[...] / cnt
    acc = jnp.dot(a_lo, wlo_ref[...], preferred_element_type=jnp.float32)
    acc += jnp.dot(a_hi, whi_ref[...], preferred_element_type=jnp.float32)
    acc += jnp.dot(x_ref[...], wr_ref[...], preferred_element_type=jnp.float32)
    acc += b_ref[...]
    o_ref[...] = jnp.maximum(acc, 0.0)


def _tc_combine(sums_lo, sums_hi, cnt16, x, wlo_t, whi_t, wr_t, b2d):
    grid = (N_NODES // ROW_BLOCK,)
    return pl.pallas_call(
        _tc_body,
        grid=grid,
        in_specs=[
            pl.BlockSpec((ROW_BLOCK, HALF), lambda i: (i, 0)),
            pl.BlockSpec((ROW_BLOCK, HALF), lambda i: (i, 0)),
            pl.BlockSpec((ROW_BLOCK, 1), lambda i: (i, 0)),
            pl.BlockSpec((ROW_BLOCK, D_IN), lambda i: (i, 0)),
            pl.BlockSpec((HALF, D_OUT), lambda i: (0, 0)),
            pl.BlockSpec((HALF, D_OUT), lambda i: (0, 0)),
            pl.BlockSpec((D_IN, D_OUT), lambda i: (0, 0)),
            pl.BlockSpec((1, D_OUT), lambda i: (0, 0)),
        ],
        out_specs=pl.BlockSpec((ROW_BLOCK, D_OUT), lambda i: (i, 0)),
        out_shape=jax.ShapeDtypeStruct((N_NODES, D_OUT), jnp.float32),
    )(sums_lo, sums_hi, cnt16, x, wlo_t, whi_t, wr_t, b2d)


def kernel(x, edge_index, W_l, b_l, W_r):
    n_edges = edge_index.shape[1]
    pad = E_PAD - n_edges
    src = edge_index[0].astype(jnp.int32)
    dst = edge_index[1].astype(jnp.int32)
    src_pad = jnp.concatenate([src, jnp.zeros((pad,), jnp.int32)])
    dst_pad = jnp.concatenate([dst, jnp.full((pad,), DUMMY_ROW, jnp.int32)])
    src_all = jnp.concatenate([src_pad, src_pad + N_NODES])
    x_stack = jnp.concatenate([x[:, :HALF], x[:, HALF:]], axis=0)

    dst_t = dst_pad.reshape(E_PAD // HALF, HALF).T
    cnt2d = _tc_hist(dst_t)
    cnt_col = cnt2d.reshape(-1)[:N_NODES].reshape(N_NODES, 1)
    out_stack = _sc_aggregate(x_stack, src_all, dst_pad)
    sums_lo = out_stack[:N_NODES]
    sums_hi = out_stack[ACC_ROWS:ACC_ROWS + N_NODES]

    wlo_t = W_l[:, :HALF].T
    whi_t = W_l[:, HALF:].T
    wr_t = W_r.T
    b2d = b_l.reshape(1, D_OUT)
    return _tc_combine(sums_lo, sums_hi, cnt_col, x, wlo_t, whi_t, wr_t, b2d)


# speedup vs baseline: 3.9487x; 1.0877x over previous
import dataclasses
import functools

import jax
import jax.numpy as jnp
from jax import lax
from jax.experimental import pallas as pl
from jax.experimental.pallas import tpu as pltpu
from jax.experimental.pallas import tpu_sc as plsc

N_NODES = 10000
D_IN = 256
D_OUT = 256
HALF = 128
LANES = 16

NC = 2
NS = 16

CHUNK = 16
N_CHUNKS = 640
E_PAD = NS * N_CHUNKS * CHUNK
N_DGROUPS = E_PAD // CHUNK
DUMMY_ROW = N_NODES
SLICE = 632
ACC_ROWS = NS * SLICE
HIST_CB = 128

ROW_BLOCK = 1000


def _sc_aggregate(x_stack, src_all, dst_flat):
    mesh = plsc.VectorSubcoreMesh(core_axis_name="c", subcore_axis_name="s")
    cp = pltpu.CompilerParams()
    if "needs_layout_passes" in pltpu.CompilerParams.__dataclass_fields__:
        cp = dataclasses.replace(cp, needs_layout_passes=False)

    @functools.partial(
        pl.kernel,
        mesh=mesh,
        compiler_params=cp,
        out_type=jax.ShapeDtypeStruct((2 * ACC_ROWS, HALF), jnp.float32),
        scratch_types=(
            [pltpu.VMEM((2, CHUNK), jnp.int32)] * 8 +
            [pltpu.VMEM((CHUNK, HALF), jnp.float32)] * 4 +
            [pltpu.VMEM_SHARED((ACC_ROWS, HALF), jnp.float32)] +
            [pltpu.SemaphoreType.DMA] * 16
        ),
    )
    def k(x_hbm, src_hbm, dst_hbm, out_hbm,
          sd0, sd1, sd2, sd3, sd4, sd5, sd6, sd7,
          rows0, rows1, rows2, rows3, acc_s,
          sg0, sg1, sg2, sg3, ss0, ss1, ss2, ss3,
          si0, si1, si2, si3, si4, si5, si6, si7):
        sd = (sd0, sd1, sd2, sd3, sd4, sd5, sd6, sd7)
        rows = (rows0, rows1, rows2, rows3)
        sg = (sg0, sg1, sg2, sg3)
        ss = (ss0, ss1, ss2, ss3)
        si = (si0, si1, si2, si3, si4, si5, si6, si7)
        c = lax.axis_index("c")
        s = lax.axis_index("s")
        base = pl.multiple_of(s * SLICE, 8)
        obase = pl.multiple_of(c * ACC_ROWS + base, 8)
        per_w = N_CHUNKS * CHUNK
        src_base = c * E_PAD + s * per_w
        dst_base = s * per_w

        def idx_issue(jj, off_dyn):
            pltpu.async_copy(src_hbm.at[pl.ds(src_base + off_dyn, CHUNK)],
                             sd[jj].at[0], si[jj])
            pltpu.async_copy(dst_hbm.at[pl.ds(dst_base + off_dyn, CHUNK)],
                             sd[jj].at[1], si[jj])

        def idx_wait(jj):
            pltpu.make_async_copy(src_hbm.at[pl.ds(src_base, CHUNK)],
                                  sd[jj].at[0], si[jj]).wait()
            pltpu.make_async_copy(dst_hbm.at[pl.ds(dst_base, CHUNK)],
                                  sd[jj].at[1], si[jj]).wait()

        def gather_issue(jj, p):
            pltpu.async_copy(x_hbm.at[sd[jj].at[0]], rows[p], sg[p])

        def gather_wait(jj, p):
            pltpu.make_async_copy(x_hbm.at[sd[jj].at[0]], rows[p],
                                  sg[p]).wait()

        def scatter_issue(jj, p):
            pltpu.async_copy(rows[p], acc_s.at[sd[jj].at[1]], ss[p], add=True)

        def scatter_wait(jj, p):
            pltpu.make_async_copy(rows[p], acc_s.at[sd[jj].at[1]],
                                  ss[p]).wait()

        zero16 = jnp.zeros((LANES,), jnp.float32)

        @pl.loop(0, CHUNK)
        def _(i):
            @pl.loop(0, HALF, step=LANES)
            def _(j):
                rows0[i, pl.ds(j, LANES)] = zero16

        for t in range(SLICE // CHUNK):
            pltpu.sync_copy(rows0, acc_s.at[pl.ds(base + t * CHUNK, CHUNK)])
        rem = SLICE % CHUNK
        pltpu.sync_copy(rows0.at[pl.ds(0, rem)],
                        acc_s.at[pl.ds(base + SLICE - rem, rem)])

        plsc.subcore_barrier()

        for j in range(4):
            idx_issue(j, j * CHUNK)
        idx_wait(0)
        gather_issue(0, 0)
        idx_wait(1)
        gather_issue(1, 1)

        @pl.loop(0, N_CHUNKS // 8)
        def _(it):
            t0 = it * 8
            for j in range(8):
                p = j % 4
                t_dyn = t0 + j
                gather_wait(j, p)
                scatter_issue(j, p)

                @pl.when(t_dyn + 4 < N_CHUNKS)
                def _(jj=(j + 4) % 8, t_dyn=t_dyn):
                    idx_issue(jj, (t_dyn + 4) * CHUNK)

                @pl.when(t_dyn + 2 < N_CHUNKS)
                def _(j=j, t_dyn=t_dyn):
                    @pl.when(t_dyn >= 2)
                    def _():
                        scatter_wait((j + 6) % 8, (j + 2) % 4)

                    idx_wait((j + 2) % 8)
                    gather_issue((j + 2) % 8, (j + 2) % 4)

        scatter_wait(4, 0)
        scatter_wait(5, 1)
        scatter_wait(6, 2)
        scatter_wait(7, 3)

        plsc.subcore_barrier()

        pltpu.sync_copy(acc_s.at[pl.ds(base, SLICE)],
                        out_hbm.at[pl.ds(obase, SLICE)])

    return k(x_stack, src_all, dst_flat)


def _hist_body(dst_ref, o_ref):
    @pl.when(pl.program_id(0) == 0)
    def _():
        o_ref[...] = jnp.zeros((HALF, HALF), jnp.float32)

    blk = dst_ref[...]
    lane = jax.lax.broadcasted_iota(jnp.int32, (1, HALF), 1)
    acc = jnp.zeros((HALF, HALF), jnp.float32)
    for b in range(HIST_CB):
        col = blk[:, b:b + 1]
        ohl = (jnp.bitwise_and(col, HALF - 1) == lane).astype(jnp.float32)
        ohh = (jax.lax.shift_right_logical(col, 7) == lane).astype(jnp.float32)
        acc += jax.lax.dot_general(ohh, ohl, (((0,), (0,)), ((), ())),
                                   preferred_element_type=jnp.float32)
    o_ref[...] += acc


def _tc_hist(dst_t):
    grid = (E_PAD // HALF // HIST_CB,)
    return pl.pallas_call(
        _hist_body,
        grid=grid,
        in_specs=[pl.BlockSpec((HALF, HIST_CB), lambda i: (0, i))],
        out_specs=pl.BlockSpec((HALF, HALF), lambda i: (0, 0)),
        out_shape=jax.ShapeDtypeStruct((HALF, HALF), jnp.float32),
    )(dst_t)


def _tc_body(slo_ref, shi_ref, cnt_ref, x_ref, wlo_ref, whi_ref, wr_ref,
             b_ref, o_ref):
    cnt = jnp.maximum(cnt_ref[...], 1.0)
    a_lo = slo_ref[...] / cnt
    a_hi = shi_ref[...] / cnt
    acc = jnp.dot(a_lo, wlo_ref[...], preferred_element_type=jnp.float32)
    acc += jnp.dot(a_hi, whi_ref[...], preferred_element_type=jnp.float32)
    acc += jnp.dot(x_ref[...], wr_ref[...], preferred_element_type=jnp.float32)
    acc += b_ref[...]
    o_ref[...] = jnp.maximum(acc, 0.0)


def _tc_combine(sums_lo, sums_hi, cnt16, x, wlo_t, whi_t, wr_t, b2d):
    grid = (N_NODES // ROW_BLOCK,)
    return pl.pallas_call(
        _tc_body,
        grid=grid,
        in_specs=[
            pl.BlockSpec((ROW_BLOCK, HALF), lambda i: (i, 0)),
            pl.BlockSpec((ROW_BLOCK, HALF), lambda i: (i, 0)),
            pl.BlockSpec((ROW_BLOCK, 1), lambda i: (i, 0)),
            pl.BlockSpec((ROW_BLOCK, D_IN), lambda i: (i, 0)),
            pl.BlockSpec((HALF, D_OUT), lambda i: (0, 0)),
            pl.BlockSpec((HALF, D_OUT), lambda i: (0, 0)),
            pl.BlockSpec((D_IN, D_OUT), lambda i: (0, 0)),
            pl.BlockSpec((1, D_OUT), lambda i: (0, 0)),
        ],
        out_specs=pl.BlockSpec((ROW_BLOCK, D_OUT), lambda i: (i, 0)),
        out_shape=jax.ShapeDtypeStruct((N_NODES, D_OUT), jnp.float32),
    )(sums_lo, sums_hi, cnt16, x, wlo_t, whi_t, wr_t, b2d)


def kernel(x, edge_index, W_l, b_l, W_r):
    n_edges = edge_index.shape[1]
    pad = E_PAD - n_edges
    src = edge_index[0].astype(jnp.int32)
    dst = edge_index[1].astype(jnp.int32)
    src_pad = jnp.concatenate([src, jnp.zeros((pad,), jnp.int32)])
    dst_pad = jnp.concatenate([dst, jnp.full((pad,), DUMMY_ROW, jnp.int32)])
    src_all = jnp.concatenate([src_pad, src_pad + N_NODES])
    x_stack = jnp.concatenate([x[:, :HALF], x[:, HALF:]], axis=0)

    dst_t = dst_pad.reshape(E_PAD // HALF, HALF).T
    cnt2d = _tc_hist(dst_t)
    cnt_col = cnt2d.reshape(-1)[:N_NODES].reshape(N_NODES, 1)
    out_stack = _sc_aggregate(x_stack, src_all, dst_pad)
    sums_lo = out_stack[:N_NODES]
    sums_hi = out_stack[ACC_ROWS:ACC_ROWS + N_NODES]

    wlo_t = W_l[:, :HALF].T
    whi_t = W_l[:, HALF:].T
    wr_t = W_r.T
    b2d = b_l.reshape(1, D_OUT)
    return _tc_combine(sums_lo, sums_hi, cnt_col, x, wlo_t, whi_t, wr_t, b2d)


# speedup vs baseline: 4.0185x; 1.0177x over previous
import dataclasses
import functools

import jax
import jax.numpy as jnp
from jax import lax
from jax.experimental import pallas as pl
from jax.experimental.pallas import tpu as pltpu
from jax.experimental.pallas import tpu_sc as plsc

N_NODES = 10000
D_IN = 256
D_OUT = 256
HALF = 128
LANES = 16

NC = 2
NS = 16

CHUNK = 16
N_CHUNKS = 640
GROUP = 8
E_PAD = NS * N_CHUNKS * CHUNK
N_DGROUPS = E_PAD // CHUNK
DUMMY_ROW = N_NODES
SLICE = 632
ACC_ROWS = NS * SLICE
HIST_CB = 128

ROW_BLOCK = 1000


def _sc_aggregate(x_stack, packed_idx):
    mesh = plsc.VectorSubcoreMesh(core_axis_name="c", subcore_axis_name="s")
    cp = pltpu.CompilerParams()
    if "needs_layout_passes" in pltpu.CompilerParams.__dataclass_fields__:
        cp = dataclasses.replace(cp, needs_layout_passes=False)

    @functools.partial(
        pl.kernel,
        mesh=mesh,
        compiler_params=cp,
        out_type=jax.ShapeDtypeStruct((2 * ACC_ROWS, HALF), jnp.float32),
        scratch_types=(
            [pltpu.VMEM((2, CHUNK), jnp.int32)] * 4 +
            [pltpu.VMEM((CHUNK, HALF), jnp.float32)] * 4 +
            [pltpu.VMEM((2, GROUP * CHUNK), jnp.int32)] +
            [pltpu.VMEM_SHARED((ACC_ROWS, HALF), jnp.float32)] +
            [pltpu.SemaphoreType.DMA] * 10
        ),
    )
    def k(x_hbm, pk_hbm, out_hbm,
          sd0, sd1, sd2, sd3, rows0, rows1, rows2, rows3, pbuf, acc_s,
          sg0, sg1, sg2, sg3, ss0, ss1, ss2, ss3, si0, si1):
        sd = (sd0, sd1, sd2, sd3)
        rows = (rows0, rows1, rows2, rows3)
        sg = (sg0, sg1, sg2, sg3)
        ss = (ss0, ss1, ss2, ss3)
        si = (si0, si1)
        c = lax.axis_index("c")
        s = lax.axis_index("s")
        base = pl.multiple_of(s * SLICE, 8)
        obase = pl.multiple_of(c * ACC_ROWS + base, 8)
        per_w = N_CHUNKS * CHUNK
        pk_base = s * per_w
        core_off = c * N_NODES

        GC = GROUP * CHUNK

        def pk_issue(q, off_dyn):
            pltpu.async_copy(pk_hbm.at[pl.ds(pk_base + off_dyn, GC)],
                             pbuf.at[q], si[q])

        def pk_wait(q):
            pltpu.make_async_copy(pk_hbm.at[pl.ds(pk_base, GC)],
                                  pbuf.at[q], si[q]).wait()

        def unpack(q, ku, slot):
            val = pbuf[q, pl.ds(ku * CHUNK, CHUNK)]
            sd[slot][0, pl.ds(0, CHUNK)] = (
                jnp.bitwise_and(val, 0xFFFF) + core_off)
            sd[slot][1, pl.ds(0, CHUNK)] = jax.lax.shift_right_logical(val, 16)

        def gather_issue(slot):
            pltpu.async_copy(x_hbm.at[sd[slot].at[0]], rows[slot], sg[slot])

        def gather_wait(slot):
            pltpu.make_async_copy(x_hbm.at[sd[slot].at[0]], rows[slot],
                                  sg[slot]).wait()

        def scatter_issue(slot):
            pltpu.async_copy(rows[slot], acc_s.at[sd[slot].at[1]], ss[slot],
                             add=True)

        def scatter_wait(slot):
            pltpu.make_async_copy(rows[slot], acc_s.at[sd[slot].at[1]],
                                  ss[slot]).wait()

        zero16 = jnp.zeros((LANES,), jnp.float32)

        @pl.loop(0, CHUNK)
        def _(i):
            @pl.loop(0, HALF, step=LANES)
            def _(j):
                rows0[i, pl.ds(j, LANES)] = zero16

        for t in range(SLICE // CHUNK):
            pltpu.sync_copy(rows0, acc_s.at[pl.ds(base + t * CHUNK, CHUNK)])
        rem = SLICE % CHUNK
        if rem:
            pltpu.sync_copy(rows0.at[pl.ds(0, rem)],
                            acc_s.at[pl.ds(base + SLICE - rem, rem)])

        plsc.subcore_barrier()

        pk_issue(0, 0)
        pk_wait(0)
        unpack(0, 0, 0)
        gather_issue(0)
        unpack(0, 1, 1)
        gather_issue(1)

        @pl.loop(0, N_CHUNKS // 16)
        def _(it):
            t0 = it * 16
            for j in range(16):
                slot = j % 4
                ku = j % 8
                q = (j // 8) % 2
                t_dyn = t0 + j
                gather_wait(slot)
                scatter_issue(slot)

                if j == 0:

                    @pl.when(t_dyn + 8 < N_CHUNKS)
                    def _(t_dyn=t_dyn):
                        pk_issue(1, (t_dyn + 8) * CHUNK)

                if j == 8:

                    @pl.when(t_dyn + 8 < N_CHUNKS)
                    def _(t_dyn=t_dyn):
                        pk_issue(0, (t_dyn + 8) * CHUNK)

                @pl.when(t_dyn + 2 < N_CHUNKS)
                def _(j=j, slot=slot, t_dyn=t_dyn):
                    @pl.when(t_dyn >= 2)
                    def _():
                        scatter_wait((slot + 2) % 4)

                    if j == 6:
                        pk_wait(1)
                    if j == 14:
                        pk_wait(0)
                    unpack(((j + 2) // 8) % 2, (j + 2) % 8, (slot + 2) % 4)
                    gather_issue((slot + 2) % 4)

        scatter_wait(0)
        scatter_wait(1)
        scatter_wait(2)
        scatter_wait(3)

        plsc.subcore_barrier()

        pltpu.sync_copy(acc_s.at[pl.ds(base, SLICE)],
                        out_hbm.at[pl.ds(obase, SLICE)])

    return k(x_stack, packed_idx)


def _hist_body(dst_ref, o_ref):
    @pl.when(pl.program_id(0) == 0)
    def _():
        o_ref[...] = jnp.zeros((HALF, HALF), jnp.float32)

    blk = dst_ref[...]
    lane = jax.lax.broadcasted_iota(jnp.int32, (1, HALF), 1)
    acc = jnp.zeros((HALF, HALF), jnp.float32)
    for b in range(HIST_CB):
        col = blk[:, b:b + 1]
        ohl = (jnp.bitwise_and(col, HALF - 1) == lane).astype(jnp.float32)
        ohh = (jax.lax.shift_right_logical(col, 7) == lane).astype(jnp.float32)
        acc += jax.lax.dot_general(ohh, ohl, (((0,), (0,)), ((), ())),
                                   preferred_element_type=jnp.float32)
    o_ref[...] += acc


def _tc_hist(dst_t):
    grid = (E_PAD // HALF // HIST_CB,)
    return pl.pallas_call(
        _hist_body,
        grid=grid,
        in_specs=[pl.BlockSpec((HALF, HIST_CB), lambda i: (0, i))],
        out_specs=pl.BlockSpec((HALF, HALF), lambda i: (0, 0)),
        out_shape=jax.ShapeDtypeStruct((HALF, HALF), jnp.float32),
    )(dst_t)


def _tc_body(slo_ref, shi_ref, cnt_ref, x_ref, wlo_ref, whi_ref, wr_ref,
             b_ref, o_ref):
    cnt = jnp.maximum(cnt_ref[...], 1.0)
    a_lo = slo_ref[...] / cnt
    a_hi = shi_ref[...] / cnt
    acc = jnp.dot(a_lo, wlo_ref[...], preferred_element_type=jnp.float32)
    acc += jnp.dot(a_hi, whi_ref[...], preferred_element_type=jnp.float32)
    acc += jnp.dot(x_ref[...], wr_ref[...], preferred_element_type=jnp.float32)
    acc += b_ref[...]
    o_ref[...] = jnp.maximum(acc, 0.0)


def _tc_combine(sums_lo, sums_hi, cnt16, x, wlo_t, whi_t, wr_t, b2d):
    grid = (N_NODES // ROW_BLOCK,)
    return pl.pallas_call(
        _tc_body,
        grid=grid,
        in_specs=[
            pl.BlockSpec((ROW_BLOCK, HALF), lambda i: (i, 0)),
            pl.BlockSpec((ROW_BLOCK, HALF), lambda i: (i, 0)),
            pl.BlockSpec((ROW_BLOCK, 1), lambda i: (i, 0)),
            pl.BlockSpec((ROW_BLOCK, D_IN), lambda i: (i, 0)),
            pl.BlockSpec((HALF, D_OUT), lambda i: (0, 0)),
            pl.BlockSpec((HALF, D_OUT), lambda i: (0, 0)),
            pl.BlockSpec((D_IN, D_OUT), lambda i: (0, 0)),
            pl.BlockSpec((1, D_OUT), lambda i: (0, 0)),
        ],
        out_specs=pl.BlockSpec((ROW_BLOCK, D_OUT), lambda i: (i, 0)),
        out_shape=jax.ShapeDtypeStruct((N_NODES, D_OUT), jnp.float32),
    )(sums_lo, sums_hi, cnt16, x, wlo_t, whi_t, wr_t, b2d)


def kernel(x, edge_index, W_l, b_l, W_r):
    n_edges = edge_index.shape[1]
    pad = E_PAD - n_edges
    src = edge_index[0].astype(jnp.int32)
    dst = edge_index[1].astype(jnp.int32)
    src_pad = jnp.concatenate([src, jnp.zeros((pad,), jnp.int32)])
    dst_pad = jnp.concatenate([dst, jnp.full((pad,), DUMMY_ROW, jnp.int32)])
    packed_idx = jnp.bitwise_or(src_pad,
                                jax.lax.shift_left(dst_pad, 16))
    x_stack = jnp.concatenate([x[:, :HALF], x[:, HALF:]], axis=0)

    dst_t = dst_pad.reshape(E_PAD // HALF, HALF).T
    cnt2d = _tc_hist(dst_t)
    cnt_col = cnt2d.reshape(-1)[:N_NODES].reshape(N_NODES, 1)
    out_stack = _sc_aggregate(x_stack, packed_idx)
    sums_lo = out_stack[:N_NODES]
    sums_hi = out_stack[ACC_ROWS:ACC_ROWS + N_NODES]

    wlo_t = W_l[:, :HALF].T
    whi_t = W_l[:, HALF:].T
    wr_t = W_r.T
    b2d = b_l.reshape(1, D_OUT)
    return _tc_combine(sums_lo, sums_hi, cnt_col, x, wlo_t, whi_t, wr_t, b2d)
